# baseline (speedup 1.0000x reference)
import functools

import jax
import jax.numpy as jnp
from jax import lax
from jax.experimental import pallas as pl
from jax.experimental.pallas import tpu as pltpu
from jax.experimental.pallas import tpu_sc as plsc

NC = 2
NS = 16
NW = NC * NS
CHUNK = 128
BLK = 2000


def _mesh():
    return plsc.VectorSubcoreMesh(core_axis_name="c", subcore_axis_name="s")


def _worker_chunks(wid, base_cnt, nx):
    base = base_cnt * wid + 2 * jnp.maximum(wid - nx, 0)
    count = base_cnt + 2 * (wid >= nx).astype(jnp.int32)
    return base, count


def _deg_kernel(n, base_cnt, nx):
    @functools.partial(
        pl.kernel,
        out_type=jax.ShapeDtypeStruct((NC, n), jnp.float32),
        mesh=_mesh(),
        scratch_types=[
            pltpu.VMEM((CHUNK,), jnp.int32),
            pltpu.VMEM((CHUNK,), jnp.int32),
            pltpu.VMEM((CHUNK,), jnp.float32),
            pltpu.VMEM_SHARED((n,), jnp.float32),
            pltpu.SemaphoreType.DMA,
            pltpu.SemaphoreType.DMA,
            pltpu.SemaphoreType.DMA,
            pltpu.SemaphoreType.DMA,
        ],
    )
    def deg(ei, zeros1, out, ib0, ib1, ones_v, acc, is0, is1, ws0, ws1):
        c = lax.axis_index("c")
        s = lax.axis_index("s")
        wid = s * NC + c
        base, count = _worker_chunks(wid, base_cnt, nx)
        for i in range(CHUNK // 16):
            ones_v[pl.ds(i * 16, 16)] = jnp.full((16,), 1.0, jnp.float32)

        @pl.when(s == 0)
        def _():
            pltpu.sync_copy(zeros1, acc)

        plsc.subcore_barrier()
        ibs = (ib0, ib1)
        sems = (is0, is1)

        def dst_chunk(k):
            return ei.at[1, pl.ds((base + k) * CHUNK, CHUNK)]

        pltpu.async_copy(dst_chunk(0), ib0, is0)
        wsems = (ws0, ws1)

        def pair(k2, carry):
            for j in range(2):
                k = k2 * 2 + j
                o = 1 - j
                ib, sem = ibs[j], sems[j]
                pltpu.make_async_copy(dst_chunk(k), ib, sem).wait()
                pltpu.async_copy(ones_v, acc.at[ib], wsems[j], add=True)

                @pl.when(k >= 1)
                def _():
                    pltpu.make_async_copy(ones_v, acc.at[ibs[o]],
                                          wsems[o]).wait()

                @pl.when(k + 1 < count)
                def _():
                    pltpu.async_copy(dst_chunk(k + 1), ibs[o], sems[o])

            return carry

        lax.fori_loop(0, count // 2, pair, 0)
        pltpu.make_async_copy(ones_v, acc.at[ib1], wsems[1]).wait()
        plsc.subcore_barrier()

        @pl.when(s == 0)
        def _():
            pltpu.sync_copy(acc, out.at[c])

    return deg


def _edge_kernel(n, d, base_cnt, nx):
    @functools.partial(
        pl.kernel,
        out_type=jax.ShapeDtypeStruct((NC, n, d), jnp.float32),
        mesh=_mesh(),
        scratch_types=[
            pltpu.VMEM((CHUNK,), jnp.int32),
            pltpu.VMEM((CHUNK,), jnp.int32),
            pltpu.VMEM((CHUNK,), jnp.int32),
            pltpu.VMEM((CHUNK,), jnp.int32),
            pltpu.VMEM((CHUNK, d), jnp.float32),
            pltpu.VMEM((CHUNK, d), jnp.float32),
            pltpu.VMEM_SHARED((n, d), jnp.float32),
            pltpu.SemaphoreType.DMA,
            pltpu.SemaphoreType.DMA,
            pltpu.SemaphoreType.DMA,
            pltpu.SemaphoreType.DMA,
            pltpu.SemaphoreType.DMA,
            pltpu.SemaphoreType.DMA,
            pltpu.SemaphoreType.DMA,
            pltpu.SemaphoreType.DMA,
        ],
    )
    def edge(ei, g_hbm, zeros2, out, sib0, sib1, dib0, dib1, rows0, rows1,
             acc, ss0, ss1, ds0, ds1, gs0, gs1, ws0, ws1):
        c = lax.axis_index("c")
        s = lax.axis_index("s")
        wid = s * NC + c
        base, count = _worker_chunks(wid, base_cnt, nx)
        rpt = n // 10

        @pl.when(s < 10)
        def _():
            pltpu.sync_copy(zeros2.at[pl.ds(s * rpt, rpt)],
                            acc.at[pl.ds(s * rpt, rpt)])

        plsc.subcore_barrier()
        sibs = (sib0, sib1)
        dibs = (dib0, dib1)
        ssems = (ss0, ss1)
        dsems = (ds0, ds1)
        rows = (rows0, rows1)
        gsems = (gs0, gs1)

        def src_chunk(k):
            return ei.at[0, pl.ds((base + k) * CHUNK, CHUNK)]

        def dst_chunk(k):
            return ei.at[1, pl.ds((base + k) * CHUNK, CHUNK)]

        pltpu.async_copy(src_chunk(0), sib0, ss0)
        pltpu.async_copy(src_chunk(1), sib1, ss1)
        pltpu.async_copy(dst_chunk(0), dib0, ds0)
        pltpu.make_async_copy(src_chunk(0), sib0, ss0).wait()
        pltpu.async_copy(g_hbm.at[sib0], rows0, gs0)
        wsems = (ws0, ws1)

        def pair(k2, carry):
            for j in range(2):
                k = k2 * 2 + j
                o = 1 - j
                pltpu.make_async_copy(g_hbm.at[sibs[j]], rows[j],
                                      gsems[j]).wait()
                pltpu.make_async_copy(dst_chunk(k), dibs[j], dsems[j]).wait()
                pltpu.async_copy(rows[j], acc.at[dibs[j]], wsems[j], add=True)

                @pl.when(k + 2 < count)
                def _():
                    pltpu.async_copy(src_chunk(k + 2), sibs[j], ssems[j])

                @pl.when(k >= 1)
                def _():
                    pltpu.make_async_copy(rows[o], acc.at[dibs[o]],
                                          wsems[o]).wait()

                @pl.when(k + 1 < count)
                def _():
                    pltpu.async_copy(dst_chunk(k + 1), dibs[o], dsems[o])
                    pltpu.make_async_copy(src_chunk(k + 1), sibs[o],
                                          ssems[o]).wait()
                    pltpu.async_copy(g_hbm.at[sibs[o]], rows[o], gsems[o])

            return carry

        lax.fori_loop(0, count // 2, pair, 0)
        pltpu.make_async_copy(rows1, acc.at[dib1], wsems[1]).wait()
        plsc.subcore_barrier()

        @pl.when(s < 10)
        def _():
            pltpu.sync_copy(acc.at[pl.ds(s * rpt, rpt)],
                            out.at[c, pl.ds(s * rpt, rpt)])

    return edge


def _scale_body(xr, wr, degr, gr):
    dinv = lax.rsqrt(degr[0] + degr[1] + 1.0)
    h = jnp.dot(xr[...], wr[...], preferred_element_type=jnp.float32)
    gr[...] = h * dinv


def _scale_kernel(n, d):
    return pl.pallas_call(
        _scale_body,
        grid=(n // BLK,),
        in_specs=[
            pl.BlockSpec((BLK, d), lambda i: (i, 0)),
            pl.BlockSpec((d, d), lambda i: (0, 0)),
            pl.BlockSpec((NC, BLK, 1), lambda i: (0, i, 0)),
        ],
        out_specs=pl.BlockSpec((BLK, d), lambda i: (i, 0)),
        out_shape=jax.ShapeDtypeStruct((n, d), jnp.float32),
    )


def _combine_body(pr, gr, degr, br, outr):
    dinv = lax.rsqrt(degr[0] + degr[1] + 1.0)
    acc = pr[0] + pr[1] + gr[...]
    outr[...] = jnp.maximum(dinv * acc + br[...], 0.0)


def _combine_kernel(n, d):
    return pl.pallas_call(
        _combine_body,
        grid=(n // BLK,),
        in_specs=[
            pl.BlockSpec((NC, BLK, d), lambda i: (0, i, 0)),
            pl.BlockSpec((BLK, d), lambda i: (i, 0)),
            pl.BlockSpec((NC, BLK, 1), lambda i: (0, i, 0)),
            pl.BlockSpec((1, d), lambda i: (0, 0)),
        ],
        out_specs=pl.BlockSpec((BLK, d), lambda i: (i, 0)),
        out_shape=jax.ShapeDtypeStruct((n, d), jnp.float32),
    )


def kernel(x, edge_index, W, b):
    n, d = x.shape
    e = edge_index.shape[1]
    assert e % CHUNK == 0 and n % BLK == 0
    chunks = e // CHUNK
    base_cnt = (chunks // NW) & ~1
    extra = chunks - base_cnt * NW
    assert extra % 2 == 0 and extra // 2 <= NW
    nx = NW - extra // 2
    zeros1 = jnp.zeros((n,), jnp.float32)
    zeros2 = jnp.zeros((n, d), jnp.float32)

    degp = _deg_kernel(n, base_cnt, nx)(edge_index, zeros1)
    degp3 = degp.reshape(NC, n, 1)
    g = _scale_kernel(n, d)(x, W, degp3)
    p = _edge_kernel(n, d, base_cnt, nx)(edge_index, g, zeros2)
    return _combine_kernel(n, d)(p, g, degp3, b.reshape(1, d))


# speedup vs baseline: 1.0739x; 1.0739x over previous
import functools

import jax
import jax.numpy as jnp
from jax import lax
from jax.experimental import pallas as pl
from jax.experimental.pallas import tpu as pltpu
from jax.experimental.pallas import tpu_sc as plsc

NC = 2
NS = 16
NW = NC * NS
CHUNK = 128
BLK = 2000


def _mesh():
    return plsc.VectorSubcoreMesh(core_axis_name="c", subcore_axis_name="s")


def _worker_chunks(wid, base_cnt, nx):
    base = base_cnt * wid + 2 * jnp.maximum(wid - nx, 0)
    count = base_cnt + 2 * (wid >= nx).astype(jnp.int32)
    return base, count


def _deg_kernel(n, base_cnt, nx):
    @functools.partial(
        pl.kernel,
        out_type=jax.ShapeDtypeStruct((NC, n), jnp.float32),
        mesh=_mesh(),
        scratch_types=[
            pltpu.VMEM((CHUNK,), jnp.int32),
            pltpu.VMEM((CHUNK,), jnp.int32),
            pltpu.VMEM((CHUNK,), jnp.float32),
            pltpu.VMEM_SHARED((n,), jnp.float32),
            pltpu.SemaphoreType.DMA,
            pltpu.SemaphoreType.DMA,
        ],
    )
    def deg(ei, zeros1, out, ib0, ib1, ones_v, acc, is0, is1):
        c = lax.axis_index("c")
        s = lax.axis_index("s")
        wid = s * NC + c
        base, count = _worker_chunks(wid, base_cnt, nx)
        for i in range(CHUNK // 16):
            ones_v[pl.ds(i * 16, 16)] = jnp.full((16,), 1.0, jnp.float32)

        @pl.when(s == 0)
        def _():
            pltpu.sync_copy(zeros1, acc)

        plsc.subcore_barrier()
        ibs = (ib0, ib1)
        sems = (is0, is1)

        def dst_chunk(k):
            return ei.at[1, pl.ds((base + k) * CHUNK, CHUNK)]

        pltpu.async_copy(dst_chunk(0), ib0, is0)
        pltpu.async_copy(dst_chunk(1), ib1, is1)

        def pair(k2, carry):
            for j in range(2):
                k = k2 * 2 + j
                ib, sem = ibs[j], sems[j]
                pltpu.make_async_copy(dst_chunk(k), ib, sem).wait()
                pltpu.sync_copy(ones_v, acc.at[ib], add=True)

                @pl.when(k + 2 < count)
                def _():
                    pltpu.async_copy(dst_chunk(k + 2), ib, sem)

            return carry

        lax.fori_loop(0, count // 2, pair, 0)
        plsc.subcore_barrier()

        @pl.when(s == 0)
        def _():
            pltpu.sync_copy(acc, out.at[c])

    return deg


def _edge_kernel(n, d, base_cnt, nx):
    @functools.partial(
        pl.kernel,
        out_type=jax.ShapeDtypeStruct((NC, n, d), jnp.float32),
        mesh=_mesh(),
        scratch_types=[
            pltpu.VMEM((CHUNK,), jnp.int32),
            pltpu.VMEM((CHUNK,), jnp.int32),
            pltpu.VMEM((CHUNK,), jnp.int32),
            pltpu.VMEM((CHUNK,), jnp.int32),
            pltpu.VMEM((CHUNK, d), jnp.float32),
            pltpu.VMEM((CHUNK, d), jnp.float32),
            pltpu.VMEM_SHARED((n, d), jnp.float32),
            pltpu.SemaphoreType.DMA,
            pltpu.SemaphoreType.DMA,
            pltpu.SemaphoreType.DMA,
            pltpu.SemaphoreType.DMA,
            pltpu.SemaphoreType.DMA,
            pltpu.SemaphoreType.DMA,
            pltpu.SemaphoreType.DMA,
            pltpu.SemaphoreType.DMA,
        ],
    )
    def edge(ei, g_hbm, zeros2, out, sib0, sib1, dib0, dib1, rows0, rows1,
             acc, ss0, ss1, ds0, ds1, gs0, gs1, ws0, ws1):
        c = lax.axis_index("c")
        s = lax.axis_index("s")
        wid = s * NC + c
        base, count = _worker_chunks(wid, base_cnt, nx)
        rpt = n // 10

        @pl.when(s < 10)
        def _():
            pltpu.sync_copy(zeros2.at[pl.ds(s * rpt, rpt)],
                            acc.at[pl.ds(s * rpt, rpt)])

        plsc.subcore_barrier()
        sibs = (sib0, sib1)
        dibs = (dib0, dib1)
        ssems = (ss0, ss1)
        dsems = (ds0, ds1)
        rows = (rows0, rows1)
        gsems = (gs0, gs1)

        def src_chunk(k):
            return ei.at[0, pl.ds((base + k) * CHUNK, CHUNK)]

        def dst_chunk(k):
            return ei.at[1, pl.ds((base + k) * CHUNK, CHUNK)]

        pltpu.async_copy(src_chunk(0), sib0, ss0)
        pltpu.async_copy(src_chunk(1), sib1, ss1)
        pltpu.async_copy(dst_chunk(0), dib0, ds0)
        pltpu.make_async_copy(src_chunk(0), sib0, ss0).wait()
        pltpu.async_copy(g_hbm.at[sib0], rows0, gs0)
        wsems = (ws0, ws1)

        def pair(k2, carry):
            for j in range(2):
                k = k2 * 2 + j
                o = 1 - j
                pltpu.make_async_copy(g_hbm.at[sibs[j]], rows[j],
                                      gsems[j]).wait()
                pltpu.make_async_copy(dst_chunk(k), dibs[j], dsems[j]).wait()
                pltpu.async_copy(rows[j], acc.at[dibs[j]], wsems[j], add=True)

                @pl.when(k + 2 < count)
                def _():
                    pltpu.async_copy(src_chunk(k + 2), sibs[j], ssems[j])

                @pl.when(k >= 1)
                def _():
                    pltpu.make_async_copy(rows[o], acc.at[dibs[o]],
                                          wsems[o]).wait()

                @pl.when(k + 1 < count)
                def _():
                    pltpu.async_copy(dst_chunk(k + 1), dibs[o], dsems[o])
                    pltpu.make_async_copy(src_chunk(k + 1), sibs[o],
                                          ssems[o]).wait()
                    pltpu.async_copy(g_hbm.at[sibs[o]], rows[o], gsems[o])

            return carry

        lax.fori_loop(0, count // 2, pair, 0)
        pltpu.make_async_copy(rows1, acc.at[dib1], wsems[1]).wait()
        plsc.subcore_barrier()

        @pl.when(s < 10)
        def _():
            pltpu.sync_copy(acc.at[pl.ds(s * rpt, rpt)],
                            out.at[c, pl.ds(s * rpt, rpt)])

    return edge


def _scale_body(xr, wr, degr, gr):
    dinv = lax.rsqrt(degr[0] + degr[1] + 1.0)
    h = jnp.dot(xr[...], wr[...], preferred_element_type=jnp.float32)
    gr[...] = h * dinv


def _scale_kernel(n, d):
    return pl.pallas_call(
        _scale_body,
        grid=(n // BLK,),
        in_specs=[
            pl.BlockSpec((BLK, d), lambda i: (i, 0)),
            pl.BlockSpec((d, d), lambda i: (0, 0)),
            pl.BlockSpec((NC, BLK, 1), lambda i: (0, i, 0)),
        ],
        out_specs=pl.BlockSpec((BLK, d), lambda i: (i, 0)),
        out_shape=jax.ShapeDtypeStruct((n, d), jnp.float32),
    )


def _combine_body(pr, gr, degr, br, outr):
    dinv = lax.rsqrt(degr[0] + degr[1] + 1.0)
    acc = pr[0] + pr[1] + gr[...]
    outr[...] = jnp.maximum(dinv * acc + br[...], 0.0)


def _combine_kernel(n, d):
    return pl.pallas_call(
        _combine_body,
        grid=(n // BLK,),
        in_specs=[
            pl.BlockSpec((NC, BLK, d), lambda i: (0, i, 0)),
            pl.BlockSpec((BLK, d), lambda i: (i, 0)),
            pl.BlockSpec((NC, BLK, 1), lambda i: (0, i, 0)),
            pl.BlockSpec((1, d), lambda i: (0, 0)),
        ],
        out_specs=pl.BlockSpec((BLK, d), lambda i: (i, 0)),
        out_shape=jax.ShapeDtypeStruct((n, d), jnp.float32),
    )


def kernel(x, edge_index, W, b):
    n, d = x.shape
    e = edge_index.shape[1]
    assert e % CHUNK == 0 and n % BLK == 0
    chunks = e // CHUNK
    base_cnt = (chunks // NW) & ~1
    extra = chunks - base_cnt * NW
    assert extra % 2 == 0 and extra // 2 <= NW
    nx = NW - extra // 2
    zeros1 = jnp.zeros((n,), jnp.float32)
    zeros2 = jnp.zeros((n, d), jnp.float32)

    degp = _deg_kernel(n, base_cnt, nx)(edge_index, zeros1)
    degp3 = degp.reshape(NC, n, 1)
    g = _scale_kernel(n, d)(x, W, degp3)
    p = _edge_kernel(n, d, base_cnt, nx)(edge_index, g, zeros2)
    return _combine_kernel(n, d)(p, g, degp3, b.reshape(1, d))
